# baseline (speedup 1.0000x reference)
import functools

import jax
import jax.numpy as jnp
from jax import lax
from jax.experimental import pallas as pl
from jax.experimental.pallas import tpu as pltpu
from jax.experimental.pallas import tpu_sc as plsc

S = 2048
D = 1024
DFF = 4096
E = 8
BLK = 256
NB = S // BLK + E - 1
NPAD = NB * BLK
FF = 2048
F = DFF // FF


def _router_body(x_ref, wr_ref, lg_ref, ei_ref, xs_ref):
    x = x_ref[...]
    lg = jnp.dot(x, wr_ref[...], preferred_element_type=jnp.float32)
    lmax = jnp.max(lg, axis=1, keepdims=True)
    ssum = jnp.sum(jnp.exp(lg - lmax), axis=1, keepdims=True)
    prob = 1.0 / ssum
    iota = lax.broadcasted_iota(jnp.int32, lg.shape, 1)
    ei = jnp.min(jnp.where(lg >= lmax, iota, E), axis=1)
    lg_ref[...] = lg
    ei_ref[...] = ei.reshape(1, 1, BLK)
    xs_ref[...] = x * prob


def _router(x2d, wr):
    n_t = S // BLK
    return pl.pallas_call(
        _router_body,
        grid=(n_t,),
        in_specs=[
            pl.BlockSpec((BLK, D), lambda t: (t, 0)),
            pl.BlockSpec((D, E), lambda t: (0, 0)),
        ],
        out_specs=[
            pl.BlockSpec((BLK, E), lambda t: (t, 0)),
            pl.BlockSpec((1, 1, BLK), lambda t: (t, 0, 0)),
            pl.BlockSpec((BLK, D), lambda t: (t, 0)),
        ],
        out_shape=[
            jax.ShapeDtypeStruct((S, E), jnp.float32),
            jax.ShapeDtypeStruct((n_t, 1, BLK), jnp.int32),
            jax.ShapeDtypeStruct((S, D), jnp.float32),
        ],
    )(x2d, wr)


def _gmm_body(be_ref, na_ref, x_ref, wi_ref, wo_ref, y_ref):
    f = pl.program_id(0)
    b = pl.program_id(1)

    @pl.when(b < na_ref[0])
    def _():
        xb = x_ref[...].astype(jnp.bfloat16)
        h = jnp.maximum(
            jnp.dot(xb, wi_ref[0], preferred_element_type=jnp.float32), 0.0)
        part = jnp.dot(h.astype(jnp.bfloat16), wo_ref[0],
                       preferred_element_type=jnp.float32)
        rows = pl.ds(b * BLK, BLK)

        @pl.when(f == 0)
        def _():
            y_ref[rows, :] = part

        @pl.when(f > 0)
        def _():
            y_ref[rows, :] += part


def _gmm(be, na, x_pad, wi, wo):
    grid_spec = pltpu.PrefetchScalarGridSpec(
        num_scalar_prefetch=2,
        grid=(F, NB),
        in_specs=[
            pl.BlockSpec((BLK, D), lambda f, b, be, na: (b, 0)),
            pl.BlockSpec((1, D, FF), lambda f, b, be, na: (be[b], 0, f)),
            pl.BlockSpec((1, FF, D), lambda f, b, be, na: (be[b], f, 0)),
        ],
        out_specs=pl.BlockSpec((NPAD, D), lambda f, b, be, na: (0, 0)),
    )
    return pl.pallas_call(
        _gmm_body,
        grid_spec=grid_spec,
        out_shape=jax.ShapeDtypeStruct((NPAD, D), jnp.float32),
    )(be, na, x_pad, wi.astype(jnp.bfloat16), wo.astype(jnp.bfloat16))


_NW = 32
_SC_MESH = plsc.VectorSubcoreMesh(core_axis_name="c", subcore_axis_name="s")


def _make_sc_row_gather(n_rows, n_out):
    rpw = n_out // _NW
    chunks = []
    off = 0
    while off < rpw:
        c = min(64, rpw - off)
        chunks.append((off, c))
        off += c

    scratch = []
    for _, c in chunks:
        scratch += [pltpu.VMEM((c,), jnp.int32),
                    pltpu.VMEM((c, D), jnp.float32),
                    pltpu.SemaphoreType.DMA]

    @functools.partial(
        pl.kernel,
        out_type=jax.ShapeDtypeStruct((n_out, D), jnp.float32),
        mesh=_SC_MESH,
        scratch_types=scratch,
    )
    def gather_k(table_hbm, idx_hbm, out_hbm, *bufs):
        wid = lax.axis_index("s") * 2 + lax.axis_index("c")
        base = wid * rpw
        copies = []
        for i, (off, c) in enumerate(chunks):
            idx_v, rows_v, sem = bufs[3 * i], bufs[3 * i + 1], bufs[3 * i + 2]
            pltpu.sync_copy(idx_hbm.at[pl.ds(base + off, c)], idx_v)
            copies.append(pltpu.async_copy(table_hbm.at[idx_v], rows_v, sem))
        for i, (off, c) in enumerate(chunks):
            rows_v = bufs[3 * i + 1]
            copies[i].wait()
            pltpu.sync_copy(rows_v, out_hbm.at[pl.ds(base + off, c)])

    return gather_k


_gather_dispatch = _make_sc_row_gather(S, NPAD)
_gather_return = _make_sc_row_gather(NPAD, S)


def _route_metadata(eflat):
    counts = jnp.bincount(eflat, length=E)
    nb_e = (counts + BLK - 1) // BLK
    cum_nb = jnp.cumsum(nb_e)
    na = cum_nb[-1].astype(jnp.int32)
    block_base = (cum_nb - nb_e) * BLK
    oneh = (eflat[:, None] == jnp.arange(E)[None, :]).astype(jnp.int32)
    rank = jnp.take_along_axis(jnp.cumsum(oneh, axis=0) - 1,
                               eflat[:, None], axis=1)[:, 0]
    dest = (block_base[eflat] + rank).astype(jnp.int32)
    gidx = (jnp.arange(NPAD, dtype=jnp.int32) % S).at[dest].set(
        jnp.arange(S, dtype=jnp.int32))
    be = jnp.searchsorted(cum_nb, jnp.arange(NB), side='right').astype(jnp.int32)
    be = jnp.minimum(be, E - 1)
    be = jnp.where(jnp.arange(NB) < na, be, be[jnp.maximum(na - 1, 0)])
    return be, na.reshape(1), gidx, dest


def kernel(hidden_states, Wr, Wi, Wo):
    x2d = hidden_states.reshape(S, D)
    logits, ei_blocks, xs = _router(x2d, Wr)
    eflat = ei_blocks.reshape(S)
    be, na, gidx, dest = _route_metadata(eflat)

    x_pad = _gather_dispatch(xs, gidx)
    y_pad = _gmm(be, na, x_pad, Wi, Wo)
    out = _gather_return(y_pad, dest)

    return (out.reshape(1, S, D),
            logits.reshape(1, S, E),
            eflat.reshape(1, S))


# speedup vs baseline: 1.4858x; 1.4858x over previous
import functools

import jax
import jax.numpy as jnp
from jax import lax
from jax.experimental import pallas as pl
from jax.experimental.pallas import tpu as pltpu
from jax.experimental.pallas import tpu_sc as plsc

S = 2048
D = 1024
DFF = 4096
E = 8
BLK = 256
NB = S // BLK + E - 1
NPAD = NB * BLK
FF = 2048
F = DFF // FF


def _router_body(x_ref, wr_ref, lg_ref, ei_ref, xs_ref):
    x = x_ref[...]
    lg = jnp.dot(x, wr_ref[...], preferred_element_type=jnp.float32)
    lmax = jnp.max(lg, axis=1, keepdims=True)
    ssum = jnp.sum(jnp.exp(lg - lmax), axis=1, keepdims=True)
    prob = 1.0 / ssum
    iota = lax.broadcasted_iota(jnp.int32, lg.shape, 1)
    ei = jnp.min(jnp.where(lg >= lmax, iota, E), axis=1)
    lg_ref[...] = lg
    ei_ref[...] = ei.reshape(1, 1, BLK)
    xs_ref[...] = x * prob


def _router(x2d, wr):
    n_t = S // BLK
    return pl.pallas_call(
        _router_body,
        grid=(n_t,),
        in_specs=[
            pl.BlockSpec((BLK, D), lambda t: (t, 0)),
            pl.BlockSpec((D, E), lambda t: (0, 0)),
        ],
        out_specs=[
            pl.BlockSpec((BLK, E), lambda t: (t, 0)),
            pl.BlockSpec((1, 1, BLK), lambda t: (t, 0, 0)),
            pl.BlockSpec((BLK, D), lambda t: (t, 0)),
        ],
        out_shape=[
            jax.ShapeDtypeStruct((S, E), jnp.float32),
            jax.ShapeDtypeStruct((n_t, 1, BLK), jnp.int32),
            jax.ShapeDtypeStruct((S, D), jnp.float32),
        ],
    )(x2d, wr)


def _gmm_body(be_ref, na_ref, x_ref, wi_ref, wo_ref, y_ref):
    f = pl.program_id(0)
    b = pl.program_id(1)

    @pl.when(b < na_ref[0])
    def _():
        h = jnp.maximum(
            jnp.dot(x_ref[...], wi_ref[0], preferred_element_type=jnp.float32), 0.0)
        part = jnp.dot(h, wo_ref[0], preferred_element_type=jnp.float32)
        rows = pl.ds(b * BLK, BLK)

        @pl.when(f == 0)
        def _():
            y_ref[rows, :] = part

        @pl.when(f > 0)
        def _():
            y_ref[rows, :] += part


def _gmm(be, na, x_pad, wi, wo):
    grid_spec = pltpu.PrefetchScalarGridSpec(
        num_scalar_prefetch=2,
        grid=(F, NB),
        in_specs=[
            pl.BlockSpec((BLK, D), lambda f, b, be, na: (b, 0)),
            pl.BlockSpec((1, D, FF), lambda f, b, be, na: (be[b], 0, f)),
            pl.BlockSpec((1, FF, D), lambda f, b, be, na: (be[b], f, 0)),
        ],
        out_specs=pl.BlockSpec((NPAD, D), lambda f, b, be, na: (0, 0)),
    )
    return pl.pallas_call(
        _gmm_body,
        grid_spec=grid_spec,
        out_shape=jax.ShapeDtypeStruct((NPAD, D), jnp.float32),
    )(be, na, x_pad, wi, wo)


_NW = 32
_SC_MESH = plsc.VectorSubcoreMesh(core_axis_name="c", subcore_axis_name="s")


def _make_sc_row_gather(n_rows, n_out):
    rpw = n_out // _NW
    chunks = []
    off = 0
    while off < rpw:
        c = min(64, rpw - off)
        chunks.append((off, c))
        off += c

    scratch = []
    for _, c in chunks:
        scratch += [pltpu.VMEM((c,), jnp.int32),
                    pltpu.VMEM((c, D), jnp.float32),
                    pltpu.SemaphoreType.DMA]

    @functools.partial(
        pl.kernel,
        out_type=jax.ShapeDtypeStruct((n_out, D), jnp.float32),
        mesh=_SC_MESH,
        scratch_types=scratch,
    )
    def gather_k(table_hbm, idx_hbm, out_hbm, *bufs):
        wid = lax.axis_index("s") * 2 + lax.axis_index("c")
        base = wid * rpw
        copies = []
        for i, (off, c) in enumerate(chunks):
            idx_v, rows_v, sem = bufs[3 * i], bufs[3 * i + 1], bufs[3 * i + 2]
            pltpu.sync_copy(idx_hbm.at[pl.ds(base + off, c)], idx_v)
            copies.append(pltpu.async_copy(table_hbm.at[idx_v], rows_v, sem))
        for i, (off, c) in enumerate(chunks):
            rows_v = bufs[3 * i + 1]
            copies[i].wait()
            pltpu.sync_copy(rows_v, out_hbm.at[pl.ds(base + off, c)])

    return gather_k


def _make_sc_row_scatter(n_src, n_out):
    rpw = n_src // _NW

    @functools.partial(
        pl.kernel,
        out_type=jax.ShapeDtypeStruct((n_out, D), jnp.float32),
        mesh=_SC_MESH,
        scratch_types=[
            pltpu.VMEM((rpw,), jnp.int32),
            pltpu.VMEM((rpw, D), jnp.float32),
            pltpu.SemaphoreType.DMA,
        ],
    )
    def scatter_k(src_hbm, idx_hbm, out_hbm, idx_v, rows_v, sem):
        wid = lax.axis_index("s") * 2 + lax.axis_index("c")
        base = wid * rpw
        pltpu.sync_copy(idx_hbm.at[pl.ds(base, rpw)], idx_v)
        pltpu.sync_copy(src_hbm.at[pl.ds(base, rpw)], rows_v)
        pltpu.async_copy(rows_v, out_hbm.at[idx_v], sem).wait()

    return scatter_k


_scatter_dispatch = _make_sc_row_scatter(S, NPAD)
_gather_return = _make_sc_row_gather(NPAD, S)


def _route_metadata(eflat):
    counts = jnp.bincount(eflat, length=E)
    nb_e = (counts + BLK - 1) // BLK
    cum_nb = jnp.cumsum(nb_e)
    na = cum_nb[-1].astype(jnp.int32)
    block_base = (cum_nb - nb_e) * BLK
    oneh = (eflat[:, None] == jnp.arange(E)[None, :]).astype(jnp.int32)
    rank = jnp.take_along_axis(jnp.cumsum(oneh, axis=0) - 1,
                               eflat[:, None], axis=1)[:, 0]
    dest = (block_base[eflat] + rank).astype(jnp.int32)
    be = jnp.searchsorted(cum_nb, jnp.arange(NB), side='right').astype(jnp.int32)
    be = jnp.minimum(be, E - 1)
    be = jnp.where(jnp.arange(NB) < na, be, be[jnp.maximum(na - 1, 0)])
    return be, na.reshape(1), dest


def kernel(hidden_states, Wr, Wi, Wo):
    x2d = hidden_states.reshape(S, D)
    logits, ei_blocks, xs = _router(x2d, Wr)
    eflat = ei_blocks.reshape(S)
    be, na, dest = _route_metadata(eflat)

    x_pad = _scatter_dispatch(xs, dest)
    y_pad = _gmm(be, na, x_pad, Wi, Wo)
    out = _gather_return(y_pad, dest)

    return (out.reshape(1, S, D),
            logits.reshape(1, S, E),
            eflat.reshape(1, S))


# speedup vs baseline: 1.5598x; 1.0498x over previous
import functools

import jax
import jax.numpy as jnp
from jax import lax
from jax.experimental import pallas as pl
from jax.experimental.pallas import tpu as pltpu
from jax.experimental.pallas import tpu_sc as plsc

S = 2048
D = 1024
DFF = 4096
E = 8
BLK = 256
NB = S // BLK + E - 1
NPAD = NB * BLK
FF = 2048
F = DFF // FF


def _router_body(x_ref, wr_ref, lg_ref, ei_ref, xs_ref):
    x = x_ref[...]
    lg = jnp.dot(x, wr_ref[...], preferred_element_type=jnp.float32)
    lmax = jnp.max(lg, axis=1, keepdims=True)
    ssum = jnp.sum(jnp.exp(lg - lmax), axis=1, keepdims=True)
    prob = 1.0 / ssum
    iota = lax.broadcasted_iota(jnp.int32, lg.shape, 1)
    ei = jnp.min(jnp.where(lg >= lmax, iota, E), axis=1)
    lg_ref[...] = lg
    ei_ref[...] = ei.reshape(1, 1, BLK)
    xs_ref[...] = x * prob


def _router(x2d, wr):
    n_t = S // BLK
    return pl.pallas_call(
        _router_body,
        grid=(n_t,),
        in_specs=[
            pl.BlockSpec((BLK, D), lambda t: (t, 0)),
            pl.BlockSpec((D, E), lambda t: (0, 0)),
        ],
        out_specs=[
            pl.BlockSpec((BLK, E), lambda t: (t, 0)),
            pl.BlockSpec((1, 1, BLK), lambda t: (t, 0, 0)),
            pl.BlockSpec((BLK, D), lambda t: (t, 0)),
        ],
        out_shape=[
            jax.ShapeDtypeStruct((S, E), jnp.float32),
            jax.ShapeDtypeStruct((n_t, 1, BLK), jnp.int32),
            jax.ShapeDtypeStruct((S, D), jnp.float32),
        ],
    )(x2d, wr)


def _gmm_body(be_ref, na_ref, x_ref, wi_ref, wo_ref, y_ref):
    f = pl.program_id(0)
    b = pl.program_id(1)

    @pl.when(b < na_ref[0])
    def _():
        h = jnp.maximum(
            jnp.dot(x_ref[...], wi_ref[0], preferred_element_type=jnp.float32), 0.0)
        part = jnp.dot(h, wo_ref[0], preferred_element_type=jnp.float32)
        rows = pl.ds(b * BLK, BLK)

        @pl.when(f == 0)
        def _():
            y_ref[rows, :] = part

        @pl.when(f > 0)
        def _():
            y_ref[rows, :] += part


def _gmm(be, na, x_pad, wi, wo):
    grid_spec = pltpu.PrefetchScalarGridSpec(
        num_scalar_prefetch=2,
        grid=(F, NB),
        in_specs=[
            pl.BlockSpec((BLK, D), lambda f, b, be, na: (b, 0)),
            pl.BlockSpec((1, D, FF), lambda f, b, be, na: (be[b], 0, f)),
            pl.BlockSpec((1, FF, D), lambda f, b, be, na: (be[b], f, 0)),
        ],
        out_specs=pl.BlockSpec((NPAD, D), lambda f, b, be, na: (0, 0)),
    )
    return pl.pallas_call(
        _gmm_body,
        grid_spec=grid_spec,
        out_shape=jax.ShapeDtypeStruct((NPAD, D), jnp.float32),
    )(be, na, x_pad, wi, wo)


_NW = 32
_SC_MESH = plsc.VectorSubcoreMesh(core_axis_name="c", subcore_axis_name="s")


def _make_sc_row_gather(n_rows, n_out):
    rpw = n_out // _NW
    chunks = []
    off = 0
    while off < rpw:
        c = min(64, rpw - off)
        chunks.append((off, c))
        off += c

    scratch = []
    for _, c in chunks:
        scratch += [pltpu.VMEM((c,), jnp.int32),
                    pltpu.VMEM((c, D), jnp.float32),
                    pltpu.SemaphoreType.DMA]

    @functools.partial(
        pl.kernel,
        out_type=jax.ShapeDtypeStruct((n_out, D), jnp.float32),
        mesh=_SC_MESH,
        scratch_types=scratch,
    )
    def gather_k(table_hbm, idx_hbm, out_hbm, *bufs):
        wid = lax.axis_index("s") * 2 + lax.axis_index("c")
        base = wid * rpw
        copies = []
        for i, (off, c) in enumerate(chunks):
            idx_v, rows_v, sem = bufs[3 * i], bufs[3 * i + 1], bufs[3 * i + 2]
            pltpu.sync_copy(idx_hbm.at[pl.ds(base + off, c)], idx_v)
            copies.append(pltpu.async_copy(table_hbm.at[idx_v], rows_v, sem))
        for i, (off, c) in enumerate(chunks):
            rows_v = bufs[3 * i + 1]
            copies[i].wait()
            pltpu.sync_copy(rows_v, out_hbm.at[pl.ds(base + off, c)])

    return gather_k


def _make_sc_row_scatter(n_src, n_out):
    rpw = n_src // _NW

    @functools.partial(
        pl.kernel,
        out_type=jax.ShapeDtypeStruct((n_out, D), jnp.float32),
        mesh=_SC_MESH,
        scratch_types=[
            pltpu.VMEM((rpw,), jnp.int32),
            pltpu.VMEM((rpw, D), jnp.float32),
            pltpu.SemaphoreType.DMA,
        ],
    )
    def scatter_k(src_hbm, idx_hbm, out_hbm, idx_v, rows_v, sem):
        wid = lax.axis_index("s") * 2 + lax.axis_index("c")
        base = wid * rpw
        pltpu.sync_copy(idx_hbm.at[pl.ds(base, rpw)], idx_v)
        pltpu.sync_copy(src_hbm.at[pl.ds(base, rpw)], rows_v)
        pltpu.async_copy(rows_v, out_hbm.at[idx_v], sem).wait()

    return scatter_k


_scatter_dispatch = _make_sc_row_scatter(S, NPAD)
_gather_return = _make_sc_row_gather(NPAD, S)


_NT = S // BLK


def _meta_body(ei_ref, dest_ref, be_ref, na_ref, counts_ref, base_ref, run_ref):
    t = pl.program_id(0)
    ei = ei_ref[0, 0, :].reshape(BLK, 1)
    oneh = (ei == lax.broadcasted_iota(jnp.int32, (BLK, E), 1))
    colsum = jnp.sum(oneh.astype(jnp.float32), axis=0, keepdims=True)

    @pl.when(t == 0)
    def _():
        counts_ref[...] = colsum

    @pl.when((t > 0) & (t < _NT))
    def _():
        counts_ref[...] += colsum

    @pl.when(t == _NT)
    def _():
        counts = counts_ref[...]
        nb = jnp.floor((counts + (BLK - 1)) * (1.0 / BLK))
        tri8 = (lax.broadcasted_iota(jnp.int32, (E, E), 0)
                <= lax.broadcasted_iota(jnp.int32, (E, E), 1)).astype(jnp.float32)
        cum = jnp.dot(nb, tri8, preferred_element_type=jnp.float32)
        base_ref[...] = (cum - nb) * BLK
        na = cum[0, E - 1]
        na_ref[...] = na.astype(jnp.int32).reshape(1, 1)
        bi = lax.broadcasted_iota(jnp.int32, (E, 16), 1).astype(jnp.float32)
        be_raw = jnp.sum((cum.reshape(E, 1) <= bi).astype(jnp.float32), axis=0,
                         keepdims=True)
        last_e = jnp.sum((cum < na).astype(jnp.float32))
        be = jnp.where(bi[:1, :] < na, jnp.minimum(be_raw, E - 1.0), last_e)
        be_ref[...] = be.astype(jnp.int32)
        run_ref[...] = jnp.zeros_like(run_ref)

    @pl.when(t >= _NT)
    def _():
        onef = oneh.astype(jnp.float32)
        tri = (lax.broadcasted_iota(jnp.int32, (BLK, BLK), 0)
               > lax.broadcasted_iota(jnp.int32, (BLK, BLK), 1)).astype(jnp.float32)
        prefix = jnp.dot(tri, onef, preferred_element_type=jnp.float32)
        slot = base_ref[...] + run_ref[...] + prefix
        dest = jnp.sum(slot * onef, axis=1)
        dest_ref[...] = dest.astype(jnp.int32).reshape(1, 1, BLK)
        run_ref[...] += colsum


def _route_metadata(ei_blocks):
    dest, be16, na = pl.pallas_call(
        _meta_body,
        grid=(2 * _NT,),
        in_specs=[pl.BlockSpec((1, 1, BLK), lambda t: (t % _NT, 0, 0))],
        out_specs=[
            pl.BlockSpec((1, 1, BLK), lambda t: (t % _NT, 0, 0)),
            pl.BlockSpec((1, 16), lambda t: (0, 0)),
            pl.BlockSpec((1, 1), lambda t: (0, 0)),
        ],
        out_shape=[
            jax.ShapeDtypeStruct((_NT, 1, BLK), jnp.int32),
            jax.ShapeDtypeStruct((1, 16), jnp.int32),
            jax.ShapeDtypeStruct((1, 1), jnp.int32),
        ],
        scratch_shapes=[
            pltpu.VMEM((1, E), jnp.float32),
            pltpu.VMEM((1, E), jnp.float32),
            pltpu.VMEM((1, E), jnp.float32),
        ],
    )(ei_blocks)
    return be16.reshape(16), na.reshape(1), dest.reshape(S)


def kernel(hidden_states, Wr, Wi, Wo):
    x2d = hidden_states.reshape(S, D)
    logits, ei_blocks, xs = _router(x2d, Wr)
    eflat = ei_blocks.reshape(S)
    be, na, dest = _route_metadata(ei_blocks)

    x_pad = _scatter_dispatch(xs, dest)
    y_pad = _gmm(be, na, x_pad, Wi, Wo)
    out = _gather_return(y_pad, dest)

    return (out.reshape(1, S, D),
            logits.reshape(1, S, E),
            eflat.reshape(1, S))
